# baseline (speedup 1.0000x reference)
import jax
import jax.numpy as jnp
from jax import lax
from jax.experimental import pallas as pl
from jax.experimental.pallas import tpu as pltpu
from jax.experimental.pallas import tpu_sc as plsc

_H = 512
_W = 512
_C = 192

_NC = 2
_NS = 16
_NW = _NC * _NS
_ROWS_PER_W = _H // _NW
_Q = 4
_P = _W // _Q
_NSTEPS = _ROWS_PER_W * _Q
_NVEC = _C // 16

_C1SQ = 0.17 * 0.17
_C2SQ = 0.29 * 0.29
_EPSSQ = 1e-16


def _worker(o_hbm, s_hbm, out_hbm, obuf, sbuf, o_sems, s_sems):
    wid = lax.axis_index("s") * _NC + lax.axis_index("c")
    row0 = wid * _ROWS_PER_W

    def in_copies(step, slot):
        row = row0 + step // _Q
        c0 = lax.rem(step, _Q) * _P
        return (
            pltpu.make_async_copy(
                o_hbm.at[0, row, pl.ds(c0, _P)], obuf.at[slot], o_sems.at[slot]
            ),
            pltpu.make_async_copy(
                s_hbm.at[0, row, pl.ds(c0, _P)], sbuf.at[slot], s_sems.at[slot]
            ),
        )

    for c in in_copies(0, 0):
        c.start()

    def step_body(step, carry):
        slot = lax.rem(step, 2)

        @pl.when(step + 1 < _NSTEPS)
        def _():
            for c in in_copies(step + 1, 1 - slot):
                c.start()

        for c in in_copies(step, slot):
            c.wait()

        row = row0 + step // _Q
        c0 = lax.rem(step, _Q) * _P

        def pixel_body(px, carry2):
            dot = jnp.zeros((16,), jnp.float32)
            n1 = jnp.zeros((16,), jnp.float32)
            n2 = jnp.zeros((16,), jnp.float32)
            for k in range(_NVEC):
                ov = obuf[slot, px, pl.ds(k * 16, 16)]
                sv = sbuf[slot, px, pl.ds(k * 16, 16)]
                dot = dot + ov * sv
                n1 = n1 + ov * ov
                n2 = n2 + sv * sv
            d = jnp.sum(dot)
            a1 = jnp.maximum(jnp.sum(n1), jnp.float32(_EPSSQ))
            a2 = jnp.maximum(jnp.sum(n2), jnp.float32(_EPSSQ))
            den = a1 * a2
            dsq = d * d
            col = c0 + px
            mask = (
                (d > 0.0)
                & (dsq > _C1SQ * den)
                & (dsq < _C2SQ * den)
                & (row > 0)
                & (col > 0)
            )

            @pl.when(mask)
            def _():
                for k in range(_NVEC):
                    obuf[slot, px, pl.ds(k * 16, 16)] = sbuf[
                        slot, px, pl.ds(k * 16, 16)
                    ]

            return carry2

        lax.fori_loop(0, _P, pixel_body, 0)
        pltpu.sync_copy(obuf.at[slot], out_hbm.at[0, row, pl.ds(c0, _P)])
        return carry

    lax.fori_loop(0, _NSTEPS, step_body, 0)


def kernel(original, styled):
    mesh = plsc.VectorSubcoreMesh(core_axis_name="c", subcore_axis_name="s")
    f = pl.kernel(
        _worker,
        out_type=jax.ShapeDtypeStruct((1, _H, _W, _C), jnp.float32),
        mesh=mesh,
        compiler_params=pltpu.CompilerParams(needs_layout_passes=False),
        scratch_types=[
            pltpu.VMEM((2, _P, _C), jnp.float32),
            pltpu.VMEM((2, _P, _C), jnp.float32),
            pltpu.SemaphoreType.DMA((2,)),
            pltpu.SemaphoreType.DMA((2,)),
        ],
    )
    return f(original, styled)


# speedup vs baseline: 1.3560x; 1.3560x over previous
import jax
import jax.numpy as jnp
from jax import lax
from jax.experimental import pallas as pl
from jax.experimental.pallas import tpu as pltpu
from jax.experimental.pallas import tpu_sc as plsc

_H = 512
_W = 512
_C = 192

_H_TC = 320
_H_SC = _H - _H_TC
_R = 16

_NC = 2
_NS = 16
_NW = _NC * _NS
_ROWS_PER_W = _H_SC // _NW
_Q = 4
_P = _W // _Q
_NSTEPS = _ROWS_PER_W * _Q
_NVEC = _C // 16

_C1SQ = 0.17 * 0.17
_C2SQ = 0.29 * 0.29
_EPSSQ = 1e-16


def _tc_block(o_ref, s_ref, out_ref):
    i = pl.program_id(0)
    o = o_ref[0]
    s = s_ref[0]
    dot = jnp.sum(o * s, axis=2, keepdims=True)
    n1 = jnp.sqrt(jnp.sum(o * o, axis=2, keepdims=True))
    n2 = jnp.sqrt(jnp.sum(s * s, axis=2, keepdims=True))
    eps = jnp.float32(1e-8)
    scores = dot / (jnp.maximum(n1, eps) * jnp.maximum(n2, eps))
    row = i * _R + lax.broadcasted_iota(jnp.int32, (_R, _W, 1), 0)
    col = lax.broadcasted_iota(jnp.int32, (_R, _W, 1), 1)
    mask = (
        (scores > 0.17)
        & (scores < 0.29)
        & (row > 0)
        & (col > 0)
    )
    out_ref[0] = jnp.where(mask, s, o)


def _tc_half(original, styled):
    return pl.pallas_call(
        _tc_block,
        grid=(_H_TC // _R,),
        in_specs=[
            pl.BlockSpec((1, _R, _W, _C), lambda i: (0, i, 0, 0)),
            pl.BlockSpec((1, _R, _W, _C), lambda i: (0, i, 0, 0)),
        ],
        out_specs=pl.BlockSpec((1, _R, _W, _C), lambda i: (0, i, 0, 0)),
        out_shape=jax.ShapeDtypeStruct((1, _H_TC, _W, _C), jnp.float32),
    )(original, styled)


def _sc_worker(o_hbm, s_hbm, out_hbm, obuf, sbuf, o_sems, s_sems):
    wid = lax.axis_index("s") * _NC + lax.axis_index("c")
    row0 = wid * _ROWS_PER_W

    def in_copies(step, slot):
        row = _H_TC + row0 + step // _Q
        c0 = lax.rem(step, _Q) * _P
        return (
            pltpu.make_async_copy(
                o_hbm.at[0, row, pl.ds(c0, _P)], obuf.at[slot], o_sems.at[slot]
            ),
            pltpu.make_async_copy(
                s_hbm.at[0, row, pl.ds(c0, _P)], sbuf.at[slot], s_sems.at[slot]
            ),
        )

    for c in in_copies(0, 0):
        c.start()

    def step_body(step, carry):
        slot = lax.rem(step, 2)

        @pl.when(step + 1 < _NSTEPS)
        def _():
            for c in in_copies(step + 1, 1 - slot):
                c.start()

        for c in in_copies(step, slot):
            c.wait()

        row = _H_TC + row0 + step // _Q
        c0 = lax.rem(step, _Q) * _P

        def pixel_body(px, carry2):
            dot = jnp.zeros((16,), jnp.float32)
            n1 = jnp.zeros((16,), jnp.float32)
            n2 = jnp.zeros((16,), jnp.float32)
            for k in range(_NVEC):
                ov = obuf[slot, px, pl.ds(k * 16, 16)]
                sv = sbuf[slot, px, pl.ds(k * 16, 16)]
                dot = dot + ov * sv
                n1 = n1 + ov * ov
                n2 = n2 + sv * sv
            d = jnp.sum(dot)
            a1 = jnp.maximum(jnp.sum(n1), jnp.float32(_EPSSQ))
            a2 = jnp.maximum(jnp.sum(n2), jnp.float32(_EPSSQ))
            den = a1 * a2
            dsq = d * d
            col = c0 + px
            mask = (
                (d > 0.0)
                & (dsq > _C1SQ * den)
                & (dsq < _C2SQ * den)
                & (row > 0)
                & (col > 0)
            )

            @pl.when(mask)
            def _():
                for k in range(_NVEC):
                    obuf[slot, px, pl.ds(k * 16, 16)] = sbuf[
                        slot, px, pl.ds(k * 16, 16)
                    ]

            return carry2

        lax.fori_loop(0, _P, pixel_body, 0)
        pltpu.sync_copy(
            obuf.at[slot],
            out_hbm.at[0, row - _H_TC, pl.ds(c0, _P)],
        )
        return carry

    lax.fori_loop(0, _NSTEPS, step_body, 0)


def _sc_half(original, styled):
    mesh = plsc.VectorSubcoreMesh(core_axis_name="c", subcore_axis_name="s")
    f = pl.kernel(
        _sc_worker,
        out_type=jax.ShapeDtypeStruct((1, _H_SC, _W, _C), jnp.float32),
        mesh=mesh,
        compiler_params=pltpu.CompilerParams(needs_layout_passes=False),
        scratch_types=[
            pltpu.VMEM((2, _P, _C), jnp.float32),
            pltpu.VMEM((2, _P, _C), jnp.float32),
            pltpu.SemaphoreType.DMA((2,)),
            pltpu.SemaphoreType.DMA((2,)),
        ],
    )
    return f(original, styled)


def kernel(original, styled):
    top = _tc_half(original, styled)
    bottom = _sc_half(original, styled)
    return jnp.concatenate([top, bottom], axis=1)


# speedup vs baseline: 1.8369x; 1.3547x over previous
import jax
import jax.numpy as jnp
from jax.experimental import pallas as pl

_H = 512
_W = 512
_C = 192
_B = 4096


def _cloak_block(o_ref, s_ref, out_ref):
    i = pl.program_id(0)
    o = o_ref[...]
    s = s_ref[...]
    dot = jnp.sum(o * s, axis=1, keepdims=True)
    n1 = jnp.sqrt(jnp.sum(o * o, axis=1, keepdims=True))
    n2 = jnp.sqrt(jnp.sum(s * s, axis=1, keepdims=True))
    eps = jnp.float32(1e-8)
    scores = dot / (jnp.maximum(n1, eps) * jnp.maximum(n2, eps))
    p = i * _B + jax.lax.broadcasted_iota(jnp.int32, (_B, 1), 0)
    mask = (
        (scores > 0.17)
        & (scores < 0.29)
        & (p >= _W)
        & ((p & (_W - 1)) != 0)
    )
    out_ref[...] = jnp.where(mask, s, o)


def kernel(original, styled):
    o2 = original.reshape(_H * _W, _C)
    s2 = styled.reshape(_H * _W, _C)
    out = pl.pallas_call(
        _cloak_block,
        grid=((_H * _W) // _B,),
        in_specs=[
            pl.BlockSpec((_B, _C), lambda i: (i, 0)),
            pl.BlockSpec((_B, _C), lambda i: (i, 0)),
        ],
        out_specs=pl.BlockSpec((_B, _C), lambda i: (i, 0)),
        out_shape=jax.ShapeDtypeStruct((_H * _W, _C), jnp.float32),
    )(o2, s2)
    return out.reshape(original.shape)
